# Initial kernel scaffold; baseline (speedup 1.0000x reference)
import jax
import jax.numpy as jnp
from jax.experimental import pallas as pl


def kernel(x, edge_index, W1, b1, W2, b2):
    raise NotImplementedError("write your pallas kernel here")


# speedup vs baseline: 63.5942x; 63.5942x over previous
import functools

import jax
import jax.numpy as jnp
from jax import lax
from jax.experimental import pallas as pl
from jax.experimental.pallas import tpu as pltpu
from jax.experimental.pallas import tpu_sc as plsc

N = 10000
E = 320000
F_IN = 128
HID = 16
C = 64

NC = 2
NS = 16
NW = NC * NS
CHUNK = 128
CPW = 80
CPD = 160
E_PAD = NW * CPW * CHUNK
N_PAD = 10240
RPT = N_PAD // NS
NB = 8


def _newton_rsqrt(v):
    i = plsc.bitcast(v, jnp.int32)
    i = jnp.int32(0x5F3759DF) - lax.shift_right_logical(i, 1)
    y = plsc.bitcast(i, jnp.float32)
    for _ in range(3):
        y = y * (1.5 - 0.5 * v * y * y)
    return y


def _split_edge_scatter(table_sh, acc_sh, src_v, dst_v, rows_v, gsem):
    def gather(g, b):
        return pltpu.async_copy(table_sh.at[src_v.at[g]], rows_v.at[b],
                                gsem.at[b])

    for b in range(NB):
        gather(b, b)

    def body(q, carry):
        for b in range(NB):
            g = q * NB + b
            pltpu.make_async_copy(table_sh.at[src_v.at[g]], rows_v.at[b],
                                  gsem.at[b]).wait()
            pltpu.sync_copy(rows_v.at[b], acc_sh.at[dst_v.at[g]], add=True)

            @pl.when(g + NB < CPW)
            def _():
                gather(g + NB, b)
        return carry

    lax.fori_loop(0, CPW // NB, body, 0)


def _sc_layer1_body(xw_hbm, zeros_hbm, zeros1_hbm, ones1_hbm,
                    src_hbm, dst_hbm,
                    s1_hbm, deg_hbm,
                    src_v, dst_v, rows_v, xwl, degl, dl, ones_v,
                    table_sh, acc_sh, acc1d, gsem, dsem, stsem):
    c = lax.axis_index("c")
    s = lax.axis_index("s")
    wid = c * NS + s

    cps = [
        pltpu.async_copy(zeros_hbm.at[pl.ds(s * RPT, RPT)],
                         acc_sh.at[pl.ds(s * RPT, RPT)], stsem.at[0]),
        pltpu.async_copy(zeros1_hbm.at[pl.ds(s * RPT, RPT)],
                         acc1d.at[pl.ds(s * RPT, RPT)], stsem.at[1]),
        pltpu.async_copy(dst_hbm.at[pl.ds(s * CPD, CPD)], dst_v,
                         stsem.at[2]),
        pltpu.async_copy(xw_hbm.at[pl.ds(s * RPT, RPT)], xwl, stsem.at[3]),
        pltpu.async_copy(ones1_hbm, ones_v, stsem.at[4]),
    ]
    for d in cps:
        d.wait()
    plsc.subcore_barrier()

    def fire(j, carry):
        pltpu.async_copy(ones_v, acc1d.at[dst_v.at[j]], dsem, add=True)
        return carry

    lax.fori_loop(0, CPD, fire, 0)

    def drain(j, carry):
        pltpu.make_async_copy(ones_v, acc1d.at[dst_v.at[j]], dsem).wait()
        return carry

    lax.fori_loop(0, CPD, drain, 0)
    plsc.subcore_barrier()

    pltpu.sync_copy(acc1d.at[pl.ds(s * RPT, RPT)], degl)

    def newton(k, carry):
        dl[pl.ds(k * 16, 16)] = _newton_rsqrt(degl[pl.ds(k * 16, 16)] + 1.0)
        return carry

    lax.fori_loop(0, RPT // 16, newton, 0)

    def scale(i, carry):
        dsp = plsc.load_gather(dl, [jnp.full((16,), i, jnp.int32)])
        xwl[i, :] = dsp * xwl[i, :]
        return carry

    lax.fori_loop(0, RPT, scale, 0)
    pltpu.sync_copy(xwl, table_sh.at[pl.ds(s * RPT, RPT)])
    d0 = pltpu.async_copy(src_hbm.at[pl.ds(wid * CPW, CPW)], src_v,
                          stsem.at[0])
    d1 = pltpu.async_copy(dst_hbm.at[pl.ds(wid * CPW, CPW)],
                          dst_v.at[pl.ds(0, CPW)], stsem.at[1])
    d0.wait()
    d1.wait()
    plsc.subcore_barrier()

    _split_edge_scatter(table_sh, acc_sh, src_v, dst_v, rows_v, gsem)
    plsc.subcore_barrier()

    pltpu.sync_copy(acc_sh.at[pl.ds(s * RPT, RPT)],
                    s1_hbm.at[c, pl.ds(s * RPT, RPT)])

    @pl.when(c == 0)
    def _():
        pltpu.sync_copy(acc1d.at[pl.ds(s * RPT, RPT)],
                        deg_hbm.at[pl.ds(s * RPT, RPT)])


@functools.cache
def _sc_layer1():
    return pl.kernel(
        _sc_layer1_body,
        mesh=plsc.VectorSubcoreMesh(core_axis_name="c", subcore_axis_name="s",
                                    num_cores=NC, num_subcores=NS),
        out_type=(
            jax.ShapeDtypeStruct((NC, N_PAD, HID), jnp.float32),
            jax.ShapeDtypeStruct((N_PAD,), jnp.float32),
        ),
        scratch_types=[
            pltpu.VMEM((CPW, CHUNK), jnp.int32),
            pltpu.VMEM((CPD, CHUNK), jnp.int32),
            pltpu.VMEM((NB, CHUNK, HID), jnp.float32),
            pltpu.VMEM((RPT, HID), jnp.float32),
            pltpu.VMEM((RPT,), jnp.float32),
            pltpu.VMEM((RPT,), jnp.float32),
            pltpu.VMEM((CHUNK,), jnp.float32),
            pltpu.VMEM_SHARED((N_PAD, HID), jnp.float32),
            pltpu.VMEM_SHARED((N_PAD, HID), jnp.float32),
            pltpu.VMEM_SHARED((N_PAD,), jnp.float32),
            pltpu.SemaphoreType.DMA((NB,)),
            pltpu.SemaphoreType.DMA,
            pltpu.SemaphoreType.DMA((5,)),
        ],
        compiler_params=pltpu.CompilerParams(use_tc_tiling_on_sc=False,
                                             needs_layout_passes=False),
    )


def _sc_layer2_body(xw_hbm, deg_hbm, s1_hbm, b1_hbm, zeros_hbm,
                    src_hbm, dst_hbm,
                    s2_hbm, z_hbm, dsp_hbm,
                    src_v, dst_v, rows_v, xwl, l0, l1, degl, dl, dspl, b1v,
                    table_sh, acc_sh, gsem, stsem):
    c = lax.axis_index("c")
    s = lax.axis_index("s")
    wid = c * NS + s

    cps = [
        pltpu.async_copy(zeros_hbm.at[pl.ds(s * RPT, RPT)],
                         acc_sh.at[pl.ds(s * RPT, RPT)], stsem.at[0]),
        pltpu.async_copy(xw_hbm.at[pl.ds(s * RPT, RPT)], xwl, stsem.at[1]),
        pltpu.async_copy(s1_hbm.at[0, pl.ds(s * RPT, RPT)], l0, stsem.at[2]),
        pltpu.async_copy(s1_hbm.at[1, pl.ds(s * RPT, RPT)], l1, stsem.at[3]),
        pltpu.async_copy(deg_hbm.at[pl.ds(s * RPT, RPT)], degl, stsem.at[4]),
        pltpu.async_copy(b1_hbm, b1v, stsem.at[5]),
        pltpu.async_copy(src_hbm.at[pl.ds(wid * CPW, CPW)], src_v,
                         stsem.at[6]),
        pltpu.async_copy(dst_hbm.at[pl.ds(wid * CPW, CPW)], dst_v,
                         stsem.at[7]),
    ]
    for d in cps:
        d.wait()

    def newton(k, carry):
        dl[pl.ds(k * 16, 16)] = _newton_rsqrt(degl[pl.ds(k * 16, 16)] + 1.0)
        return carry

    lax.fori_loop(0, RPT // 16, newton, 0)
    b1row = b1v[...]

    def zrow(i, carry):
        dsp = plsc.load_gather(dl, [jnp.full((16,), i, jnp.int32)])
        y1 = dsp * xwl[i, :]
        h = jnp.maximum(dsp * (l0[i, :] + l1[i, :] + y1) + b1row, 0.0)
        xwl[i, :] = dsp * h
        dspl[i, :] = dsp
        return carry

    lax.fori_loop(0, RPT, zrow, 0)
    pltpu.sync_copy(xwl, table_sh.at[pl.ds(s * RPT, RPT)])

    @pl.when(c == 0)
    def _():
        pltpu.sync_copy(xwl, z_hbm.at[pl.ds(s * RPT, RPT)])
        pltpu.sync_copy(dspl, dsp_hbm.at[pl.ds(s * RPT, RPT)])

    plsc.subcore_barrier()
    _split_edge_scatter(table_sh, acc_sh, src_v, dst_v, rows_v, gsem)
    plsc.subcore_barrier()
    pltpu.sync_copy(acc_sh.at[pl.ds(s * RPT, RPT)],
                    s2_hbm.at[c, pl.ds(s * RPT, RPT)])


@functools.cache
def _sc_layer2():
    return pl.kernel(
        _sc_layer2_body,
        mesh=plsc.VectorSubcoreMesh(core_axis_name="c", subcore_axis_name="s",
                                    num_cores=NC, num_subcores=NS),
        out_type=(
            jax.ShapeDtypeStruct((NC, N_PAD, HID), jnp.float32),
            jax.ShapeDtypeStruct((N_PAD, HID), jnp.float32),
            jax.ShapeDtypeStruct((N_PAD, HID), jnp.float32),
        ),
        scratch_types=[
            pltpu.VMEM((CPW, CHUNK), jnp.int32),
            pltpu.VMEM((CPW, CHUNK), jnp.int32),
            pltpu.VMEM((NB, CHUNK, HID), jnp.float32),
            pltpu.VMEM((RPT, HID), jnp.float32),
            pltpu.VMEM((RPT, HID), jnp.float32),
            pltpu.VMEM((RPT, HID), jnp.float32),
            pltpu.VMEM((RPT,), jnp.float32),
            pltpu.VMEM((RPT,), jnp.float32),
            pltpu.VMEM((RPT, HID), jnp.float32),
            pltpu.VMEM((HID,), jnp.float32),
            pltpu.VMEM_SHARED((N_PAD, HID), jnp.float32),
            pltpu.VMEM_SHARED((N_PAD, HID), jnp.float32),
            pltpu.SemaphoreType.DMA((NB,)),
            pltpu.SemaphoreType.DMA((8,)),
        ],
        compiler_params=pltpu.CompilerParams(use_tc_tiling_on_sc=False,
                                             needs_layout_passes=False),
    )


def _tc_xw_body(x_ref, w1_ref, xw_ref):
    xw_ref[:N, :] = jnp.dot(x_ref[...], w1_ref[...],
                            preferred_element_type=jnp.float32)
    xw_ref[N:, :] = jnp.zeros((N_PAD - N, HID), jnp.float32)


def _tc_out_body(s2_ref, z_ref, dsp_ref, w2_ref, b2_ref, out_ref):
    pre = dsp_ref[:N, :] * (s2_ref[0, :N, :] + s2_ref[1, :N, :]
                            + z_ref[:N, :])
    out_ref[...] = (
        jnp.dot(pre, w2_ref[...], preferred_element_type=jnp.float32)
        + b2_ref[...]
    )


def kernel(x, edge_index, W1, b1, W2, b2):
    src = edge_index[0]
    dst = edge_index[1]
    pad = E_PAD - E
    src_p = jnp.concatenate([src, jnp.zeros((pad,), src.dtype)]) \
        .astype(jnp.int32).reshape(-1, CHUNK)
    dst_p = jnp.concatenate([dst, jnp.full((pad,), N, dst.dtype)]) \
        .astype(jnp.int32).reshape(-1, CHUNK)
    zeros = jnp.zeros((N_PAD, HID), jnp.float32)
    zeros1 = jnp.zeros((N_PAD,), jnp.float32)
    ones1 = jnp.ones((CHUNK,), jnp.float32)

    xw = pl.pallas_call(
        _tc_xw_body,
        out_shape=jax.ShapeDtypeStruct((N_PAD, HID), jnp.float32),
    )(x, W1)

    s1, deg = _sc_layer1()(xw, zeros, zeros1, ones1, src_p, dst_p)

    s2, z, dsp = _sc_layer2()(xw, deg, s1, b1, zeros, src_p, dst_p)

    out = pl.pallas_call(
        _tc_out_body,
        out_shape=jax.ShapeDtypeStruct((N, C), jnp.float32),
    )(s2, z, dsp, W2, b2.reshape(1, C))
    return out


# speedup vs baseline: 86.1238x; 1.3543x over previous
import functools

import jax
import jax.numpy as jnp
from jax import lax
from jax.experimental import pallas as pl
from jax.experimental.pallas import tpu as pltpu
from jax.experimental.pallas import tpu_sc as plsc

N = 10000
E = 320000
F_IN = 128
HID = 16
C = 64

NC = 2
NS = 16
NW = NC * NS
CHUNK = 128
ROWS = E // CHUNK
WQ, WR = divmod(ROWS, NW)
TQ, TR = divmod(ROWS, NS)
N_PAD = 10240
RPT = N_PAD // NS
NB = 8
QMAX = (WQ + 1 + NB - 1) // NB


def _newton_rsqrt(v):
    i = plsc.bitcast(v, jnp.int32)
    i = jnp.int32(0x5F3759DF) - lax.shift_right_logical(i, 1)
    y = plsc.bitcast(i, jnp.float32)
    for _ in range(3):
        y = y * (1.5 - 0.5 * v * y * y)
    return y


def _split_bounds(wid):
    base = wid * WQ + jnp.minimum(wid, WR)
    cnt = WQ + (wid < WR).astype(jnp.int32)
    return base, cnt


def _stage_edges(e_hbm, ev, base, has_extra, nmain, sem0, sem1):
    d0 = pltpu.async_copy(e_hbm.at[pl.ds(base, nmain)],
                          ev.at[pl.ds(0, nmain)], sem0)

    @pl.when(has_extra)
    def _():
        pltpu.async_copy(e_hbm.at[base + nmain], ev.at[nmain], sem1).wait()

    d0.wait()


def _split_edge_scatter(table_sh, acc_sh, ev, rows_v, gsem, cnt):
    def gather(g, b):
        return pltpu.async_copy(table_sh.at[ev.at[g, 0]], rows_v.at[b],
                                gsem.at[b])

    for b in range(NB):
        gather(b, b)

    def body(q, carry):
        for b in range(NB):
            g = q * NB + b

            @pl.when(g < cnt)
            def _():
                pltpu.make_async_copy(table_sh.at[ev.at[g, 0]], rows_v.at[b],
                                      gsem.at[b]).wait()
                pltpu.sync_copy(rows_v.at[b], acc_sh.at[ev.at[g, 1]],
                                add=True)

                @pl.when(g + NB < cnt)
                def _():
                    gather(g + NB, b)
        return carry

    lax.fori_loop(0, QMAX, body, 0)


def _sc_layer1_body(xw_hbm, zeros_hbm, zeros1_hbm, ones1_hbm, e_hbm,
                    s1_hbm, deg_hbm,
                    ev, rows_v, xwl, degl, dl, ones_v,
                    table_sh, acc_sh, acc1d, gsem, dsem, stsem):
    c = lax.axis_index("c")
    s = lax.axis_index("s")
    wid = c * NS + s

    tbase = s * TQ + jnp.minimum(s, TR)
    tcnt = TQ + (s < TR).astype(jnp.int32)
    cps = [
        pltpu.async_copy(zeros_hbm.at[pl.ds(s * RPT, RPT)],
                         acc_sh.at[pl.ds(s * RPT, RPT)], stsem.at[0]),
        pltpu.async_copy(zeros1_hbm.at[pl.ds(s * RPT, RPT)],
                         acc1d.at[pl.ds(s * RPT, RPT)], stsem.at[1]),
        pltpu.async_copy(xw_hbm.at[pl.ds(s * RPT, RPT)],
                         xwl, stsem.at[3]),
        pltpu.async_copy(ones1_hbm, ones_v, stsem.at[4]),
    ]
    _stage_edges(e_hbm, ev, tbase, s < TR, TQ, stsem.at[2], stsem.at[5])
    for d in cps:
        d.wait()
    plsc.subcore_barrier()

    def fire(j, carry):
        pltpu.async_copy(ones_v, acc1d.at[ev.at[j, 1]], dsem, add=True)
        return carry

    lax.fori_loop(0, tcnt, fire, 0)

    def drain(j, carry):
        pltpu.make_async_copy(ones_v, acc1d.at[ev.at[j, 1]], dsem).wait()
        return carry

    lax.fori_loop(0, tcnt, drain, 0)
    plsc.subcore_barrier()

    pltpu.sync_copy(acc1d.at[pl.ds(s * RPT, RPT)], degl)

    def newton(k, carry):
        dl[pl.ds(k * 16, 16)] = _newton_rsqrt(degl[pl.ds(k * 16, 16)] + 1.0)
        return carry

    lax.fori_loop(0, RPT // 16, newton, 0)

    def scale(i, carry):
        dsp = plsc.load_gather(dl, [jnp.full((16,), i, jnp.int32)])
        xwl[i, :] = dsp * xwl[i, :]
        return carry

    lax.fori_loop(0, RPT, scale, 0)
    pltpu.sync_copy(xwl, table_sh.at[pl.ds(s * RPT, RPT)])
    wbase, wcnt = _split_bounds(wid)
    _stage_edges(e_hbm, ev, wbase, wid < WR, WQ, stsem.at[0], stsem.at[1])
    plsc.subcore_barrier()

    _split_edge_scatter(table_sh, acc_sh, ev, rows_v, gsem, wcnt)
    plsc.subcore_barrier()

    pltpu.sync_copy(acc_sh.at[pl.ds(s * RPT, RPT)],
                    s1_hbm.at[c, pl.ds(s * RPT, RPT)])

    @pl.when(c == 0)
    def _():
        pltpu.sync_copy(acc1d.at[pl.ds(s * RPT, RPT)],
                        deg_hbm.at[pl.ds(s * RPT, RPT)])


@functools.cache
def _sc_layer1():
    return pl.kernel(
        _sc_layer1_body,
        mesh=plsc.VectorSubcoreMesh(core_axis_name="c", subcore_axis_name="s",
                                    num_cores=NC, num_subcores=NS),
        out_type=(
            jax.ShapeDtypeStruct((NC, N_PAD, HID), jnp.float32),
            jax.ShapeDtypeStruct((N_PAD,), jnp.float32),
        ),
        scratch_types=[
            pltpu.VMEM((TQ + 1, 2, CHUNK), jnp.int32),
            pltpu.VMEM((NB, CHUNK, HID), jnp.float32),
            pltpu.VMEM((RPT, HID), jnp.float32),
            pltpu.VMEM((RPT,), jnp.float32),
            pltpu.VMEM((RPT,), jnp.float32),
            pltpu.VMEM((CHUNK,), jnp.float32),
            pltpu.VMEM_SHARED((N_PAD, HID), jnp.float32),
            pltpu.VMEM_SHARED((N_PAD, HID), jnp.float32),
            pltpu.VMEM_SHARED((N_PAD,), jnp.float32),
            pltpu.SemaphoreType.DMA((NB,)),
            pltpu.SemaphoreType.DMA,
            pltpu.SemaphoreType.DMA((6,)),
        ],
        compiler_params=pltpu.CompilerParams(use_tc_tiling_on_sc=False,
                                             needs_layout_passes=False),
    )


def _sc_layer2_body(xw_hbm, deg_hbm, s1_hbm, b1_hbm, zeros_hbm, e_hbm,
                    s2a_hbm, s2b_hbm, z_hbm, dsp_hbm,
                    ev, rows_v, xwl, l0, l1, degl, dl, dspl, b1v,
                    table_sh, acc_sh, gsem, stsem):
    c = lax.axis_index("c")
    s = lax.axis_index("s")
    wid = c * NS + s
    wbase, wcnt = _split_bounds(wid)

    cps = [
        pltpu.async_copy(zeros_hbm.at[pl.ds(s * RPT, RPT)],
                         acc_sh.at[pl.ds(s * RPT, RPT)], stsem.at[0]),
        pltpu.async_copy(xw_hbm.at[pl.ds(s * RPT, RPT)],
                         xwl, stsem.at[1]),
        pltpu.async_copy(s1_hbm.at[0, pl.ds(s * RPT, RPT)], l0, stsem.at[2]),
        pltpu.async_copy(s1_hbm.at[1, pl.ds(s * RPT, RPT)], l1, stsem.at[3]),
        pltpu.async_copy(deg_hbm.at[pl.ds(s * RPT, RPT)], degl, stsem.at[4]),
        pltpu.async_copy(b1_hbm, b1v, stsem.at[5]),
    ]
    _stage_edges(e_hbm, ev, wbase, wid < WR, WQ, stsem.at[6], stsem.at[7])
    for d in cps:
        d.wait()

    def newton(k, carry):
        dl[pl.ds(k * 16, 16)] = _newton_rsqrt(degl[pl.ds(k * 16, 16)] + 1.0)
        return carry

    lax.fori_loop(0, RPT // 16, newton, 0)
    b1row = b1v[...]

    def zrow(i, carry):
        dsp = plsc.load_gather(dl, [jnp.full((16,), i, jnp.int32)])
        y1 = dsp * xwl[i, :]
        h = jnp.maximum(dsp * (l0[i, :] + l1[i, :] + y1) + b1row, 0.0)
        xwl[i, :] = dsp * h
        dspl[i, :] = dsp
        return carry

    lax.fori_loop(0, RPT, zrow, 0)
    pltpu.sync_copy(xwl, table_sh.at[pl.ds(s * RPT, RPT)])

    @pl.when(c == 0)
    def _():
        pltpu.sync_copy(xwl,
                        z_hbm.at[pl.ds(s * RPT, RPT)])
        pltpu.sync_copy(dspl,
                        dsp_hbm.at[pl.ds(s * RPT, RPT)])

    plsc.subcore_barrier()
    _split_edge_scatter(table_sh, acc_sh, ev, rows_v, gsem, wcnt)
    plsc.subcore_barrier()

    @pl.when(c == 0)
    def _():
        pltpu.sync_copy(acc_sh.at[pl.ds(s * RPT, RPT)],
                        s2a_hbm.at[pl.ds(s * RPT, RPT)])

    @pl.when(c == 1)
    def _():
        pltpu.sync_copy(acc_sh.at[pl.ds(s * RPT, RPT)],
                        s2b_hbm.at[pl.ds(s * RPT, RPT)])


@functools.cache
def _sc_layer2():
    return pl.kernel(
        _sc_layer2_body,
        mesh=plsc.VectorSubcoreMesh(core_axis_name="c", subcore_axis_name="s",
                                    num_cores=NC, num_subcores=NS),
        out_type=(
            jax.ShapeDtypeStruct((N_PAD, HID), jnp.float32),
            jax.ShapeDtypeStruct((N_PAD, HID), jnp.float32),
            jax.ShapeDtypeStruct((N_PAD, HID), jnp.float32),
            jax.ShapeDtypeStruct((N_PAD, HID), jnp.float32),
        ),
        scratch_types=[
            pltpu.VMEM((WQ + 1, 2, CHUNK), jnp.int32),
            pltpu.VMEM((NB, CHUNK, HID), jnp.float32),
            pltpu.VMEM((RPT, HID), jnp.float32),
            pltpu.VMEM((RPT, HID), jnp.float32),
            pltpu.VMEM((RPT, HID), jnp.float32),
            pltpu.VMEM((RPT,), jnp.float32),
            pltpu.VMEM((RPT,), jnp.float32),
            pltpu.VMEM((RPT, HID), jnp.float32),
            pltpu.VMEM((HID,), jnp.float32),
            pltpu.VMEM_SHARED((N_PAD, HID), jnp.float32),
            pltpu.VMEM_SHARED((N_PAD, HID), jnp.float32),
            pltpu.SemaphoreType.DMA((NB,)),
            pltpu.SemaphoreType.DMA((8,)),
        ],
        compiler_params=pltpu.CompilerParams(use_tc_tiling_on_sc=False,
                                             needs_layout_passes=False),
    )


def _tc_xw_body(x_ref, w1_ref, xw_ref):
    xw_ref[:N, :] = jnp.dot(x_ref[...], w1_ref[...],
                            preferred_element_type=jnp.float32)
    xw_ref[N:, :] = jnp.zeros((N_PAD - N, HID), jnp.float32)


def _tc_out_body(s2a_ref, s2b_ref, z_ref, dsp_ref, w2b_ref, b2b_ref,
                 out_ref):
    pre = dsp_ref[...] * (s2a_ref[...] + s2b_ref[...] + z_ref[...])
    out_ref[...] = (
        jnp.dot(pre, w2b_ref[...], preferred_element_type=jnp.float32)
        + b2b_ref[...]
    )


def kernel(x, edge_index, W1, b1, W2, b2):
    e3 = edge_index.astype(jnp.int32).reshape(2, ROWS, CHUNK) \
        .transpose(1, 0, 2)
    zeros = jnp.zeros((N_PAD, HID), jnp.float32)
    zeros1 = jnp.zeros((N_PAD,), jnp.float32)
    ones1 = jnp.ones((CHUNK,), jnp.float32)

    W2big = jnp.kron(jnp.eye(8, dtype=jnp.float32), W2)
    b2big = jnp.tile(b2.reshape(1, C), (1, 8))

    xw = pl.pallas_call(
        _tc_xw_body,
        out_shape=jax.ShapeDtypeStruct((N_PAD, HID), jnp.float32),
    )(x, W1)

    s1, deg = _sc_layer1()(xw, zeros, zeros1, ones1, e3)

    s2a, s2b, z, dsp = _sc_layer2()(xw, deg, s1, b1, zeros, e3)

    outp = pl.pallas_call(
        _tc_out_body,
        out_shape=jax.ShapeDtypeStruct((N_PAD // 8, 8 * C), jnp.float32),
    )(s2a.reshape(N_PAD // 8, 128), s2b.reshape(N_PAD // 8, 128),
      z.reshape(N_PAD // 8, 128), dsp.reshape(N_PAD // 8, 128),
      W2big, b2big)
    return outp.reshape(N_PAD, C)[:N]


# speedup vs baseline: 89.2433x; 1.0362x over previous
import functools

import jax
import jax.numpy as jnp
from jax import lax
from jax.experimental import pallas as pl
from jax.experimental.pallas import tpu as pltpu
from jax.experimental.pallas import tpu_sc as plsc

N = 10000
E = 320000
F_IN = 128
HID = 16
C = 64

NC = 2
NS = 16
NW = NC * NS
CHUNK = 128
ROWS = E // CHUNK
WQ, WR = divmod(ROWS, NW)
TQ, TR = divmod(ROWS, NS)
N_PAD = 10240
RPT = N_PAD // NS
NB = 8
QMAX = (WQ + 1 + NB - 1) // NB


def _newton_rsqrt(v):
    i = plsc.bitcast(v, jnp.int32)
    i = jnp.int32(0x5F3759DF) - lax.shift_right_logical(i, 1)
    y = plsc.bitcast(i, jnp.float32)
    for _ in range(3):
        y = y * (1.5 - 0.5 * v * y * y)
    return y


def _split_bounds(wid):
    base = wid * WQ + jnp.minimum(wid, WR)
    cnt = WQ + (wid < WR).astype(jnp.int32)
    return base, cnt


def _stage_edges(e_hbm, ev, base, has_extra, nmain, sem0, sem1):
    d0 = pltpu.async_copy(e_hbm.at[pl.ds(base, nmain)],
                          ev.at[pl.ds(0, nmain)], sem0)

    @pl.when(has_extra)
    def _():
        pltpu.async_copy(e_hbm.at[base + nmain], ev.at[nmain], sem1).wait()

    d0.wait()


def _split_edge_scatter(table_sh, acc_sh, ev, rows_v, gsem, cnt):
    def gather(g, b):
        return pltpu.async_copy(table_sh.at[ev.at[g, 0]], rows_v.at[b],
                                gsem.at[b])

    for b in range(NB):
        gather(b, b)

    def body(q, carry):
        for b in range(NB):
            g = q * NB + b

            @pl.when(g < cnt)
            def _():
                pltpu.make_async_copy(table_sh.at[ev.at[g, 0]], rows_v.at[b],
                                      gsem.at[b]).wait()
                pltpu.sync_copy(rows_v.at[b], acc_sh.at[ev.at[g, 1]],
                                add=True)

                @pl.when(g + NB < cnt)
                def _():
                    gather(g + NB, b)
        return carry

    lax.fori_loop(0, QMAX, body, 0)


def _sc_layer1_body(xw_hbm, zeros_hbm, zeros1_hbm, ones1_hbm, e_hbm,
                    s1_hbm, deg_hbm,
                    ev, rows_v, xwl, degl, dl, ones_v,
                    table_sh, acc_sh, acc1d, gsem, dsem, stsem):
    c = lax.axis_index("c")
    s = lax.axis_index("s")
    wid = c * NS + s

    tbase = s * TQ + jnp.minimum(s, TR)
    tcnt = TQ + (s < TR).astype(jnp.int32)
    cps = [
        pltpu.async_copy(zeros_hbm.at[pl.ds(s * RPT, RPT)],
                         acc_sh.at[pl.ds(s * RPT, RPT)], stsem.at[0]),
        pltpu.async_copy(zeros1_hbm.at[pl.ds(s * RPT, RPT)],
                         acc1d.at[pl.ds(s * RPT, RPT)], stsem.at[1]),
        pltpu.async_copy(xw_hbm.at[pl.ds(s * RPT, RPT), pl.ds(0, HID)],
                         xwl, stsem.at[3]),
        pltpu.async_copy(ones1_hbm, ones_v, stsem.at[4]),
    ]
    _stage_edges(e_hbm, ev, tbase, s < TR, TQ, stsem.at[2], stsem.at[5])
    for d in cps:
        d.wait()
    plsc.subcore_barrier()

    def fire(j, carry):
        pltpu.async_copy(ones_v, acc1d.at[ev.at[j, 1]], dsem, add=True)
        return carry

    lax.fori_loop(0, tcnt, fire, 0)

    def drain(j, carry):
        pltpu.make_async_copy(ones_v, acc1d.at[ev.at[j, 1]], dsem).wait()
        return carry

    lax.fori_loop(0, tcnt, drain, 0)
    plsc.subcore_barrier()

    pltpu.sync_copy(acc1d.at[pl.ds(s * RPT, RPT)], degl)

    def newton(k, carry):
        dl[pl.ds(k * 16, 16)] = _newton_rsqrt(degl[pl.ds(k * 16, 16)] + 1.0)
        return carry

    lax.fori_loop(0, RPT // 16, newton, 0)

    def scale(i, carry):
        dsp = plsc.load_gather(dl, [jnp.full((16,), i, jnp.int32)])
        xwl[i, :] = dsp * xwl[i, :]
        return carry

    lax.fori_loop(0, RPT, scale, 0)
    pltpu.sync_copy(xwl, table_sh.at[pl.ds(s * RPT, RPT)])
    wbase, wcnt = _split_bounds(wid)
    _stage_edges(e_hbm, ev, wbase, wid < WR, WQ, stsem.at[0], stsem.at[1])
    plsc.subcore_barrier()

    _split_edge_scatter(table_sh, acc_sh, ev, rows_v, gsem, wcnt)
    plsc.subcore_barrier()

    pltpu.sync_copy(acc_sh.at[pl.ds(s * RPT, RPT)],
                    s1_hbm.at[c, pl.ds(s * RPT, RPT)])

    @pl.when(c == 0)
    def _():
        pltpu.sync_copy(acc1d.at[pl.ds(s * RPT, RPT)],
                        deg_hbm.at[pl.ds(s * RPT, RPT)])


@functools.cache
def _sc_layer1():
    return pl.kernel(
        _sc_layer1_body,
        mesh=plsc.VectorSubcoreMesh(core_axis_name="c", subcore_axis_name="s",
                                    num_cores=NC, num_subcores=NS),
        out_type=(
            jax.ShapeDtypeStruct((NC, N_PAD, HID), jnp.float32),
            jax.ShapeDtypeStruct((N_PAD,), jnp.float32),
        ),
        scratch_types=[
            pltpu.VMEM((TQ + 1, 2, CHUNK), jnp.int32),
            pltpu.VMEM((NB, CHUNK, HID), jnp.float32),
            pltpu.VMEM((RPT, HID), jnp.float32),
            pltpu.VMEM((RPT,), jnp.float32),
            pltpu.VMEM((RPT,), jnp.float32),
            pltpu.VMEM((CHUNK,), jnp.float32),
            pltpu.VMEM_SHARED((N_PAD, HID), jnp.float32),
            pltpu.VMEM_SHARED((N_PAD, HID), jnp.float32),
            pltpu.VMEM_SHARED((N_PAD,), jnp.float32),
            pltpu.SemaphoreType.DMA((NB,)),
            pltpu.SemaphoreType.DMA,
            pltpu.SemaphoreType.DMA((6,)),
        ],
        compiler_params=pltpu.CompilerParams(use_tc_tiling_on_sc=False,
                                             needs_layout_passes=False),
    )


def _sc_layer2_body(xw_hbm, deg_hbm, s1_hbm, b1_hbm, zeros_hbm, e_hbm,
                    s2a_hbm, s2b_hbm, z_hbm, dsp_hbm,
                    ev, rows_v, xwl, l0, l1, degl, dl, dspl, b1v,
                    table_sh, acc_sh, gsem, stsem):
    c = lax.axis_index("c")
    s = lax.axis_index("s")
    wid = c * NS + s
    wbase, wcnt = _split_bounds(wid)

    cps = [
        pltpu.async_copy(zeros_hbm.at[pl.ds(s * RPT, RPT)],
                         acc_sh.at[pl.ds(s * RPT, RPT)], stsem.at[0]),
        pltpu.async_copy(xw_hbm.at[pl.ds(s * RPT, RPT), pl.ds(0, HID)],
                         xwl, stsem.at[1]),
        pltpu.async_copy(s1_hbm.at[0, pl.ds(s * RPT, RPT)], l0, stsem.at[2]),
        pltpu.async_copy(s1_hbm.at[1, pl.ds(s * RPT, RPT)], l1, stsem.at[3]),
        pltpu.async_copy(deg_hbm.at[pl.ds(s * RPT, RPT)], degl, stsem.at[4]),
        pltpu.async_copy(b1_hbm, b1v, stsem.at[5]),
    ]
    _stage_edges(e_hbm, ev, wbase, wid < WR, WQ, stsem.at[6], stsem.at[7])
    for d in cps:
        d.wait()

    def newton(k, carry):
        dl[pl.ds(k * 16, 16)] = _newton_rsqrt(degl[pl.ds(k * 16, 16)] + 1.0)
        return carry

    lax.fori_loop(0, RPT // 16, newton, 0)
    b1row = b1v[...]

    def zrow(i, carry):
        dsp = plsc.load_gather(dl, [jnp.full((16,), i, jnp.int32)])
        y1 = dsp * xwl[i, :]
        h = jnp.maximum(dsp * (l0[i, :] + l1[i, :] + y1) + b1row, 0.0)
        xwl[i, :] = dsp * h
        dspl[i, :] = dsp
        return carry

    lax.fori_loop(0, RPT, zrow, 0)
    pltpu.sync_copy(xwl, table_sh.at[pl.ds(s * RPT, RPT)])

    @pl.when(c == 0)
    def _():
        pltpu.sync_copy(xwl,
                        z_hbm.at[pl.ds(s * RPT, RPT)])
        pltpu.sync_copy(dspl,
                        dsp_hbm.at[pl.ds(s * RPT, RPT)])

    plsc.subcore_barrier()
    _split_edge_scatter(table_sh, acc_sh, ev, rows_v, gsem, wcnt)
    plsc.subcore_barrier()

    @pl.when(c == 0)
    def _():
        pltpu.sync_copy(acc_sh.at[pl.ds(s * RPT, RPT)],
                        s2a_hbm.at[pl.ds(s * RPT, RPT)])

    @pl.when(c == 1)
    def _():
        pltpu.sync_copy(acc_sh.at[pl.ds(s * RPT, RPT)],
                        s2b_hbm.at[pl.ds(s * RPT, RPT)])


@functools.cache
def _sc_layer2():
    return pl.kernel(
        _sc_layer2_body,
        mesh=plsc.VectorSubcoreMesh(core_axis_name="c", subcore_axis_name="s",
                                    num_cores=NC, num_subcores=NS),
        out_type=(
            jax.ShapeDtypeStruct((N_PAD, HID), jnp.float32),
            jax.ShapeDtypeStruct((N_PAD, HID), jnp.float32),
            jax.ShapeDtypeStruct((N_PAD, HID), jnp.float32),
            jax.ShapeDtypeStruct((N_PAD, HID), jnp.float32),
        ),
        scratch_types=[
            pltpu.VMEM((WQ + 1, 2, CHUNK), jnp.int32),
            pltpu.VMEM((NB, CHUNK, HID), jnp.float32),
            pltpu.VMEM((RPT, HID), jnp.float32),
            pltpu.VMEM((RPT, HID), jnp.float32),
            pltpu.VMEM((RPT, HID), jnp.float32),
            pltpu.VMEM((RPT,), jnp.float32),
            pltpu.VMEM((RPT,), jnp.float32),
            pltpu.VMEM((RPT, HID), jnp.float32),
            pltpu.VMEM((HID,), jnp.float32),
            pltpu.VMEM_SHARED((N_PAD, HID), jnp.float32),
            pltpu.VMEM_SHARED((N_PAD, HID), jnp.float32),
            pltpu.SemaphoreType.DMA((NB,)),
            pltpu.SemaphoreType.DMA((8,)),
        ],
        compiler_params=pltpu.CompilerParams(use_tc_tiling_on_sc=False,
                                             needs_layout_passes=False),
    )


def _tc_xw_body(x_ref, w1_ref, xw_ref):
    xw = jnp.dot(x_ref[...], w1_ref[...], preferred_element_type=jnp.float32)
    xw_ref[:N, :] = jnp.pad(xw, ((0, 0), (0, 128 - HID)))
    xw_ref[N:, :] = jnp.zeros((N_PAD - N, 128), jnp.float32)


def _tc_out_body(s2a_ref, s2b_ref, z_ref, dsp_ref, w2b_ref, b2b_ref,
                 out_ref):
    pre = dsp_ref[...] * (s2a_ref[...] + s2b_ref[...] + z_ref[...])
    out_ref[...] = (
        jnp.dot(pre, w2b_ref[...], preferred_element_type=jnp.float32)
        + b2b_ref[...]
    )


def kernel(x, edge_index, W1, b1, W2, b2):
    e3 = edge_index.astype(jnp.int32).reshape(2, ROWS, CHUNK) \
        .transpose(1, 0, 2)
    zeros = jnp.zeros((N_PAD, HID), jnp.float32)
    zeros1 = jnp.zeros((N_PAD,), jnp.float32)
    ones1 = jnp.ones((CHUNK,), jnp.float32)

    W2big = jnp.kron(jnp.eye(8, dtype=jnp.float32), W2)
    b2big = jnp.tile(b2.reshape(1, C), (1, 8))

    xw = pl.pallas_call(
        _tc_xw_body,
        out_shape=jax.ShapeDtypeStruct((N_PAD, 128), jnp.float32),
    )(x, W1)

    s1, deg = _sc_layer1()(xw, zeros, zeros1, ones1, e3)

    s2a, s2b, z, dsp = _sc_layer2()(xw, deg, s1, b1, zeros, e3)

    outp = pl.pallas_call(
        _tc_out_body,
        out_shape=jax.ShapeDtypeStruct((N_PAD // 8, 8 * C), jnp.float32),
    )(s2a.reshape(N_PAD // 8, 128), s2b.reshape(N_PAD // 8, 128),
      z.reshape(N_PAD // 8, 128), dsp.reshape(N_PAD // 8, 128),
      W2big, b2big)
    return outp.reshape(N_PAD, C)[:N]
